# baseline (speedup 1.0000x reference)
import functools

import jax
import jax.numpy as jnp
from jax import lax
from jax.experimental import pallas as pl
from jax.experimental.pallas import tpu as pltpu
from jax.experimental.pallas import tpu_sc as plsc

_R = 512

_NC, _NS = 2, 16
_NW = _NC * _NS
_IDXC = 96


def _argmin_body(x_ref, cb_ref, idx_ref, csq_ref):
    cbc = cb_ref[...]

    @pl.when(pl.program_id(0) == 0)
    def _init():
        csq_ref[...] = 0.5 * jnp.sum(cbc * cbc, axis=1, keepdims=True)

    xb = x_ref[...]
    dots = lax.dot_general(cbc, xb, (((1,), (1,)), ((), ())),
                           preferred_element_type=jnp.float32)
    score = csq_ref[...] - dots
    m = jnp.min(score, axis=0, keepdims=True)
    io = lax.broadcasted_iota(jnp.int32, score.shape, 0)
    sel = jnp.where(score == m, io, jnp.int32(score.shape[0]))
    idx = jnp.min(sel, axis=0, keepdims=True)
    idx_ref[...] = idx[0]


def _compute_indices(flat, codebook):
    bt, d = flat.shape
    k = codebook.shape[0]
    out = pl.pallas_call(
        _argmin_body,
        grid=(bt // _R,),
        in_specs=[
            pl.BlockSpec((_R, d), lambda i: (i, 0)),
            pl.BlockSpec((k, d), lambda i: (0, 0)),
        ],
        out_specs=pl.BlockSpec((_R,), lambda i: (i,)),
        out_shape=jax.ShapeDtypeStruct((bt,), jnp.int32),
        scratch_shapes=[
            pltpu.VMEM((k, 1), jnp.float32),
        ],
    )(flat, codebook)
    return out


def _sc_gather(idx_flat, codebook, bt):
    d = codebook.shape[1]
    rows_w = bt // _NW
    chunks = rows_w // _IDXC
    mesh = plsc.VectorSubcoreMesh(core_axis_name="c", subcore_axis_name="s")

    @functools.partial(
        pl.kernel,
        mesh=mesh,
        out_type=jax.ShapeDtypeStruct((bt, d), jnp.float32),
        compiler_params=pltpu.CompilerParams(use_tc_tiling_on_sc=False),
        scratch_types=[
            pltpu.VMEM((rows_w,), jnp.int32),
            pltpu.VMEM((rows_w, d), jnp.float32),
            pltpu.SemaphoreType.DMA,
        ],
    )
    def gather_kernel(idx_hbm, table_hbm, out_hbm, idx_v, rows_v, sem):
        wid = lax.axis_index("s") * _NC + lax.axis_index("c")
        base = wid * rows_w
        pltpu.sync_copy(idx_hbm.at[pl.ds(base, rows_w)], idx_v)
        copies = [
            pltpu.async_copy(table_hbm.at[idx_v.at[pl.ds(j * _IDXC, _IDXC)]],
                             rows_v.at[pl.ds(j * _IDXC, _IDXC)], sem)
            for j in range(chunks)
        ]
        for c in copies:
            c.wait()
        pltpu.sync_copy(rows_v, out_hbm.at[pl.ds(base, rows_w)])

    return gather_kernel(idx_flat, codebook)


@jax.jit
def kernel(x, codebook):
    b, t, d = x.shape
    bt = b * t
    flat = x.reshape(bt, d)
    idx = _compute_indices(flat, codebook)
    y = _sc_gather(idx, codebook, bt)
    return (x, y.reshape(b, t, d))


# speedup vs baseline: 1.1904x; 1.1904x over previous
import functools

import jax
import jax.numpy as jnp
from jax import lax
from jax.experimental import pallas as pl
from jax.experimental.pallas import tpu as pltpu
from jax.experimental.pallas import tpu_sc as plsc

_R = 6144

_NC, _NS = 2, 16
_NW = _NC * _NS
_IDXC = 96


def _argmin_body(x_ref, cb_ref, idx_ref, csq_ref, io_ref):
    cbc = cb_ref[...]
    k = cbc.shape[0]

    @pl.when(pl.program_id(0) == 0)
    def _init():
        csq_ref[...] = 0.5 * jnp.sum(cbc * cbc, axis=1, keepdims=True)
        io_ref[...] = lax.broadcasted_iota(jnp.int32, (k, 1), 0).astype(
            jnp.float32)

    xb = x_ref[...]
    dots = lax.dot_general(cbc, xb, (((1,), (1,)), ((), ())),
                           preferred_element_type=jnp.float32)
    score = csq_ref[...] - dots
    m = jnp.min(score, axis=0, keepdims=True)
    sel = jnp.where(score == m, io_ref[...], jnp.float32(k))
    idx = jnp.min(sel, axis=0, keepdims=True)
    idx_ref[...] = idx[0].astype(jnp.int32)


def _compute_indices(flat, codebook):
    bt, d = flat.shape
    k = codebook.shape[0]
    out = pl.pallas_call(
        _argmin_body,
        grid=(bt // _R,),
        in_specs=[
            pl.BlockSpec((_R, d), lambda i: (i, 0)),
            pl.BlockSpec((k, d), lambda i: (0, 0)),
        ],
        out_specs=pl.BlockSpec((_R,), lambda i: (i,)),
        out_shape=jax.ShapeDtypeStruct((bt,), jnp.int32),
        scratch_shapes=[
            pltpu.VMEM((k, 1), jnp.float32),
            pltpu.VMEM((k, 1), jnp.float32),
        ],
    )(flat, codebook)
    return out


def _sc_gather(idx_flat, codebook, bt):
    d = codebook.shape[1]
    rows_w = bt // _NW
    chunks = rows_w // _IDXC
    mesh = plsc.VectorSubcoreMesh(core_axis_name="c", subcore_axis_name="s")

    @functools.partial(
        pl.kernel,
        mesh=mesh,
        out_type=jax.ShapeDtypeStruct((bt, d), jnp.float32),
        compiler_params=pltpu.CompilerParams(use_tc_tiling_on_sc=False,
                                             disable_bounds_checks=True),
        scratch_types=[
            pltpu.VMEM((rows_w,), jnp.int32),
            pltpu.VMEM((rows_w, d), jnp.float32),
            pltpu.SemaphoreType.DMA,
        ],
    )
    def gather_kernel(idx_hbm, table_hbm, out_hbm, idx_v, rows_v, sem):
        wid = lax.axis_index("s") * _NC + lax.axis_index("c")
        base = wid * rows_w
        pltpu.sync_copy(idx_hbm.at[pl.ds(base, rows_w)], idx_v)
        copies = [
            pltpu.async_copy(table_hbm.at[idx_v.at[pl.ds(j * _IDXC, _IDXC)]],
                             rows_v.at[pl.ds(j * _IDXC, _IDXC)], sem)
            for j in range(chunks)
        ]
        for c in copies:
            c.wait()
        pltpu.sync_copy(rows_v, out_hbm.at[pl.ds(base, rows_w)])

    return gather_kernel(idx_flat, codebook)


@jax.jit
def kernel(x, codebook):
    b, t, d = x.shape
    bt = b * t
    flat = x.reshape(bt, d)
    idx = _compute_indices(flat, codebook)
    y = _sc_gather(idx, codebook, bt)
    return (x, y.reshape(b, t, d))
